# baseline (speedup 1.0000x reference)
import jax, jax.numpy as jnp
from jax.experimental import pallas as pl

_B, _P, _NC, _K = 8, 1024, 40, 32
_RADII = (0.2, 0.3, 0.4)


def _fps_c(pos_b, S):
    Bq, Pq, _ = pos_b.shape
    sel0 = jnp.zeros((Bq, S), dtype=jnp.int32)
    mind0 = jnp.full((Bq, Pq), jnp.inf, dtype=pos_b.dtype)
    cur0 = jnp.zeros((Bq,), dtype=jnp.int32)
    def body(i, carry):
        sel, mind, cur = carry
        sel = sel.at[:, i].set(cur)
        pc = jnp.take_along_axis(pos_b, cur[:, None, None], axis=1)
        d = jnp.sum((pos_b - pc) ** 2, axis=-1)
        mind = jnp.minimum(mind, d)
        cur = jnp.argmax(mind, axis=1).astype(jnp.int32)
        return (sel, mind, cur)
    sel, _, _ = jax.lax.fori_loop(0, S, body, (sel0, mind0, cur0))
    return sel


def _precompute_c(pos_b):
    out = []
    p = pos_b
    for r in _RADII:
        S = p.shape[1] // 2
        sel = _fps_c(p, S)
        p_s = jnp.take_along_axis(p, sel[:, :, None], axis=1)
        d2 = jnp.sum((p_s[:, :, None, :] - p[:, None, :, :]) ** 2, axis=-1)
        neg = jnp.where(d2 <= r * r, -d2, -jnp.inf)
        vals, nidx = jax.lax.top_k(neg, _K)
        valid = vals > -jnp.inf
        out.append((sel, nidx, valid))
        p = p_s
    return out


def _mlp2_c(h, W1, b1, W2, b2):
    return jax.nn.relu(jax.nn.relu(h @ W1 + b1) @ W2 + b2)


def kernel(x, pos, batch, sa1_W1, sa1_b1, sa1_W2, sa1_b2, sa2_W1, sa2_b1, sa2_W2, sa2_b2, sa3_W1, sa3_b1, sa3_W2, sa3_b2, ga_W1, ga_b1, ga_W2, ga_b2, lin1_W, lin1_b, lin2_W, lin2_b, lin3_W, lin3_b):
    ws = (sa1_W1, sa1_b1, sa1_W2, sa1_b2, sa2_W1, sa2_b1, sa2_W2, sa2_b2, sa3_W1, sa3_b1, sa3_W2, sa3_b2, ga_W1, ga_b1, ga_W2, ga_b2, lin1_W, lin1_b, lin2_W, lin2_b, lin3_W, lin3_b)
    idxs = _precompute_c(pos.reshape(_B, _P, 3))
    h = x.reshape(_B, _P, -1)
    p = pos.reshape(_B, _P, 3)
    for li, (sel, nidx, valid) in enumerate(idxs):
        p_s = jnp.take_along_axis(p, sel[:, :, None], axis=1)
        x_n = jax.vmap(lambda a, ii: a[ii])(h, nidx)
        p_n = jax.vmap(lambda a, ii: a[ii])(p, nidx)
        rel = p_n - p_s[:, :, None, :]
        W1, b1, W2, b2 = ws[4 * li:4 * li + 4]
        msg = _mlp2_c(jnp.concatenate([x_n, rel], axis=-1), W1, b1, W2, b2)
        msg = jnp.where(valid[:, :, :, None], msg, -jnp.inf)
        h = jnp.max(msg, axis=2)
        p = p_s
    g = jnp.max(_mlp2_c(jnp.concatenate([h, p], axis=-1), ws[12], ws[13], ws[14], ws[15]), axis=1)
    h = jax.nn.relu(g @ ws[16] + ws[17])
    h = jax.nn.relu(h @ ws[18] + ws[19])
    return h @ ws[20] + ws[21]


# speedup vs baseline: 1.5634x; 1.5634x over previous
import functools
import jax
import jax.numpy as jnp
from jax.experimental import pallas as pl

B, P, NUM_CLASSES, K_NEIGH = 8, 1024, 40, 32
RADII = (0.2, 0.3, 0.4)
S1, S2, S3 = 512, 256, 128


def _fps_body(px_ref, py_ref, pz_ref,
              o1x, o1y, o1z, o2x, o2y, o2z, o3x, o3y, o3z):
    def run_level(X, Y, Z, S):
        Pn = X.shape[1]
        lane = jax.lax.broadcasted_iota(jnp.int32, (B, Pn), 1)
        laneS = jax.lax.broadcasted_iota(jnp.int32, (B, S), 1)

        def body(i, c):
            mind, cur, ax, ay, az = c
            oh = lane == cur
            pcx = jnp.sum(jnp.where(oh, X, 0.0), axis=1, keepdims=True)
            pcy = jnp.sum(jnp.where(oh, Y, 0.0), axis=1, keepdims=True)
            pcz = jnp.sum(jnp.where(oh, Z, 0.0), axis=1, keepdims=True)
            selm = laneS == i
            ax = jnp.where(selm, pcx, ax)
            ay = jnp.where(selm, pcy, ay)
            az = jnp.where(selm, pcz, az)
            dx = X - pcx
            dy = Y - pcy
            dz = Z - pcz
            d = (dx * dx + dy * dy) + dz * dz
            mind = jnp.minimum(mind, d)
            m = jnp.max(mind, axis=1, keepdims=True)
            cur = jnp.min(jnp.where(mind == m, lane, Pn), axis=1,
                          keepdims=True).astype(jnp.int32)
            return (mind, cur, ax, ay, az)

        init = (jnp.full((B, Pn), jnp.inf, jnp.float32),
                jnp.zeros((B, 1), jnp.int32),
                jnp.zeros((B, S), jnp.float32),
                jnp.zeros((B, S), jnp.float32),
                jnp.zeros((B, S), jnp.float32))
        _, _, ax, ay, az = jax.lax.fori_loop(0, S, body, init)
        return ax, ay, az

    a1 = run_level(px_ref[...], py_ref[...], pz_ref[...], S1)
    o1x[...], o1y[...], o1z[...] = a1
    a2 = run_level(a1[0], a1[1], a1[2], S2)
    o2x[...], o2y[...], o2z[...] = a2
    a3 = run_level(a2[0], a2[1], a2[2], S3)
    o3x[...], o3y[...], o3z[...] = a3


def _fps_pallas(px, py, pz):
    outs = [jax.ShapeDtypeStruct((B, s), jnp.float32)
            for s in (S1, S1, S1, S2, S2, S2, S3, S3, S3)]
    return pl.pallas_call(_fps_body, out_shape=outs)(px, py, pz)


def _mlp2(h, W1, b1, W2, b2):
    return jax.nn.relu(jax.nn.relu(h @ W1 + b1) @ W2 + b2)


def kernel(x, pos, batch, sa1_W1, sa1_b1, sa1_W2, sa1_b2, sa2_W1, sa2_b1,
           sa2_W2, sa2_b2, sa3_W1, sa3_b1, sa3_W2, sa3_b2, ga_W1, ga_b1,
           ga_W2, ga_b2, lin1_W, lin1_b, lin2_W, lin2_b, lin3_W, lin3_b):
    ws = (sa1_W1, sa1_b1, sa1_W2, sa1_b2, sa2_W1, sa2_b1, sa2_W2, sa2_b2,
          sa3_W1, sa3_b1, sa3_W2, sa3_b2, ga_W1, ga_b1, ga_W2, ga_b2,
          lin1_W, lin1_b, lin2_W, lin2_b, lin3_W, lin3_b)
    pos3 = pos.reshape(B, P, 3)
    px, py, pz = pos3[:, :, 0], pos3[:, :, 1], pos3[:, :, 2]
    (p1x, p1y, p1z, p2x, p2y, p2z, p3x, p3y, p3z) = _fps_pallas(px, py, pz)
    p_samp = [jnp.stack([p1x, p1y, p1z], -1),
              jnp.stack([p2x, p2y, p2z], -1),
              jnp.stack([p3x, p3y, p3z], -1)]

    h = x.reshape(B, P, -1)
    p = pos3
    for li, r in enumerate(RADII):
        p_s = p_samp[li]
        d2 = jnp.sum((p_s[:, :, None, :] - p[:, None, :, :]) ** 2, axis=-1)
        neg = jnp.where(d2 <= r * r, -d2, -jnp.inf)
        vals, nidx = jax.lax.top_k(neg, K_NEIGH)
        valid = vals > -jnp.inf
        x_n = jax.vmap(lambda a, ii: a[ii])(h, nidx)
        p_n = jax.vmap(lambda a, ii: a[ii])(p, nidx)
        rel = p_n - p_s[:, :, None, :]
        W1, b1, W2, b2 = ws[4 * li:4 * li + 4]
        msg = _mlp2(jnp.concatenate([x_n, rel], axis=-1), W1, b1, W2, b2)
        msg = jnp.where(valid[:, :, :, None], msg, -jnp.inf)
        h = jnp.max(msg, axis=2)
        p = p_s
    g = jnp.max(_mlp2(jnp.concatenate([h, p], axis=-1),
                      ws[12], ws[13], ws[14], ws[15]), axis=1)
    h = jax.nn.relu(g @ ws[16] + ws[17])
    h = jax.nn.relu(h @ ws[18] + ws[19])
    return h @ ws[20] + ws[21]
